# baseline (speedup 1.0000x reference)
import jax
import jax.numpy as jnp
from jax.experimental import pallas as pl
from jax.experimental.pallas import tpu as pltpu

N = 100000
E = 1600000
H1 = 2
C1 = 32
OUT = 64


def _ln_body(h_ref, w_ref, b_ref, o_ref):
    h = h_ref[...]
    mu = jnp.mean(h, axis=-1, keepdims=True)
    var = jnp.mean((h - mu) ** 2, axis=-1, keepdims=True)
    o_ref[...] = (h - mu) * jax.lax.rsqrt(var + 1e-5) * w_ref[...] + b_ref[...]


def _layer_norm_pallas(h, w, b):
    R = 1000
    grid = (N // R,)
    return pl.pallas_call(
        _ln_body,
        grid=grid,
        in_specs=[
            pl.BlockSpec((R, OUT), lambda i: (i, 0)),
            pl.BlockSpec((1, OUT), lambda i: (0, 0)),
            pl.BlockSpec((1, OUT), lambda i: (0, 0)),
        ],
        out_specs=pl.BlockSpec((R, OUT), lambda i: (i, 0)),
        out_shape=jax.ShapeDtypeStruct((N, OUT), jnp.float32),
    )(h, w.reshape(1, OUT), b.reshape(1, OUT))


def _gat_conv(x, ei, W, a_src, a_dst, b, heads, out_ch):
    n = x.shape[0]
    h = (x @ W).reshape(n, heads, out_ch)
    alpha_s = (h * a_src[None, :, :]).sum(-1)
    alpha_d = (h * a_dst[None, :, :]).sum(-1)
    src = ei[0]
    dst = ei[1]
    e = jax.nn.leaky_relu(alpha_s[src] + alpha_d[dst], negative_slope=0.2)
    emax = jax.ops.segment_max(e, dst, num_segments=n)
    emax = jnp.where(jnp.isfinite(emax), emax, 0.0)
    ex = jnp.exp(e - emax[dst])
    denom = jax.ops.segment_sum(ex, dst, num_segments=n)
    alpha = ex / (denom[dst] + 1e-16)
    msg = h[src] * alpha[:, :, None]
    out = jax.ops.segment_sum(msg, dst, num_segments=n)
    return out.reshape(n, heads * out_ch) + b


def kernel(x, edge_index, type_ids, type_emb, W1, a_src1, a_dst1, b1, W2, a_src2, a_dst2, b2, ln_w, ln_b):
    te = type_emb[type_ids]
    xin = jnp.concatenate([x, te], axis=1)
    self_loops = jnp.stack([jnp.arange(N, dtype=edge_index.dtype)] * 2, axis=0)
    ei = jnp.concatenate([edge_index, self_loops], axis=1)
    h = _gat_conv(xin, ei, W1, a_src1, a_dst1, b1, H1, C1)
    h = jax.nn.relu(h)
    h = _gat_conv(h, ei, W2, a_src2, a_dst2, b2, 1, OUT)
    return _layer_norm_pallas(h, ln_w, ln_b)


# speedup vs baseline: 51.1232x; 51.1232x over previous
import functools
import jax
import jax.numpy as jnp
from jax import lax
from jax.experimental import pallas as pl
from jax.experimental.pallas import tpu as pltpu
from jax.experimental.pallas import tpu_sc as plsc

N = 100000
E = 1600000
NC = 2
NS = 16
NW = NC * NS
LN = 16
NPAD = 100352
EPAD = NW * 50176
EPT = EPAD // NW
K1 = 512
K2 = 1024
RPT = NPAD // NS
ZR = 196
RT = 2048
TGRID = NPAD // RT

@functools.lru_cache(maxsize=None)
def _mesh():
    return plsc.VectorSubcoreMesh(core_axis_name="c", subcore_axis_name="s",
                                  num_cores=NC, num_subcores=NS)


def _a1_body(x_ref, oh_ref, temb_ref, w1a_ref, w1b_ref, a1s_ref, a1d_ref,
             h_ref, p_ref):
    te = jnp.dot(oh_ref[...], temb_ref[...], preferred_element_type=jnp.float32)
    h = jnp.dot(x_ref[...], w1a_ref[...], preferred_element_type=jnp.float32)
    h = h + jnp.dot(te, w1b_ref[...], preferred_element_type=jnp.float32)
    h_ref[...] = h
    ts = h * a1s_ref[...]
    td = h * a1d_ref[...]
    as0 = jnp.sum(ts[:, :32], axis=1, keepdims=True)
    as1 = jnp.sum(ts[:, 32:], axis=1, keepdims=True)
    ad0 = jnp.sum(td[:, :32], axis=1, keepdims=True)
    ad1 = jnp.sum(td[:, 32:], axis=1, keepdims=True)
    v0 = as0 + ad0
    v1 = as1 + ad1
    w0 = jnp.exp(jnp.maximum(v0, 0.2 * v0))
    w1 = jnp.exp(jnp.maximum(v1, 0.2 * v1))
    z = jnp.zeros((RT, 10), jnp.float32)
    p_ref[...] = jnp.concatenate([as0, as1, ad0, ad1, w0, w1, z], axis=1)


def _a1(xp, oh, temb, w1a, w1b, a1s, a1d):
    full = lambda shp: pl.BlockSpec(shp, lambda i: tuple(0 for _ in shp))
    row = lambda m: pl.BlockSpec((RT, m), lambda i: (i, 0))
    return pl.pallas_call(
        _a1_body,
        grid=(TGRID,),
        in_specs=[row(8), row(8), full((8, 16)), full((8, 64)),
                  full((16, 64)), full((1, 64)), full((1, 64))],
        out_specs=[row(64), row(LN)],
        out_shape=[jax.ShapeDtypeStruct((NPAD, 64), jnp.float32),
                   jax.ShapeDtypeStruct((NPAD, LN), jnp.float32)],
    )(xp, oh, temb, w1a, w1b, a1s, a1d)


def _make_b1(H):
    out_type = ([jax.ShapeDtypeStruct((EPAD,), jnp.float32) for _ in range(H)]
                + [jax.ShapeDtypeStruct((NC, NPAD, LN), jnp.float32)])
    NB = K1 // 128
    scratch = ([pltpu.VMEM((NB, 128), jnp.int32),
                pltpu.VMEM((NB, 128), jnp.int32),
                pltpu.VMEM((K1, LN), jnp.float32),
                pltpu.VMEM((K1, LN), jnp.float32),
                pltpu.VMEM((K1, LN), jnp.float32)]
               + [pltpu.VMEM((K1,), jnp.float32) for _ in range(H)]
               + [pltpu.VMEM((ZR, LN), jnp.float32),
                  pltpu.VMEM_SHARED((NPAD, LN), jnp.float32),
                  pltpu.SemaphoreType.DMA])

    def body(src_hbm, dst_hbm, p_hbm, *rest):
        w_hbm = rest[:H]
        dp_hbm = rest[H]
        srcv, dstv, psrc, pdst, wrows = rest[H + 1:H + 6]
        whs = rest[H + 6:H + 6 + H]
        zbuf, den_sp, sem = rest[H + 6 + H:]
        cid = lax.axis_index("c")
        sid = lax.axis_index("s")
        wid = cid * NS + sid
        zero16 = jnp.zeros((LN,), jnp.float32)
        iota16 = lax.iota(jnp.int32, 16)

        def zrow(i, _):
            zbuf[i, :] = zero16
            return 0
        lax.fori_loop(0, ZR, zrow, 0)

        def wz(i, _):
            wrows[i, :] = zero16
            return 0
        lax.fori_loop(0, K1, wz, 0)

        for j in range(RPT // ZR):
            pltpu.sync_copy(zbuf, den_sp.at[pl.ds(sid * RPT + j * ZR, ZR)])
        plsc.subcore_barrier()

        def chunk(ci, _):
            off = wid * EPT + ci * K1
            row0 = wid * (EPT // 128) + ci * NB
            pltpu.sync_copy(src_hbm.at[pl.ds(row0, NB)], srcv)
            pltpu.sync_copy(dst_hbm.at[pl.ds(row0, NB)], dstv)
            descs = []
            for b in range(NB):
                descs.append(pltpu.async_copy(
                    p_hbm.at[srcv.at[b]], psrc.at[pl.ds(b * 128, 128)], sem))
                descs.append(pltpu.async_copy(
                    p_hbm.at[dstv.at[b]], pdst.at[pl.ds(b * 128, 128)], sem))
            for d in descs:
                d.wait()
            for h in range(H):
                colh = jnp.full((16,), h, jnp.int32)
                colad = jnp.full((16,), H + h, jnp.int32)

                def grp(g, _, h=h, colh=colh, colad=colad):
                    r = g * 16 + iota16
                    a_s = plsc.load_gather(psrc, [r, colh])
                    a_d = plsc.load_gather(pdst, [r, colad])
                    v = a_s + a_d
                    w = jnp.exp(jnp.maximum(v, 0.2 * v))
                    whs[h][pl.ds(g * 16, 16)] = w
                    plsc.store_scatter(wrows, [r, colh], w)
                    return 0
                lax.fori_loop(0, K1 // 16, grp, 0)
                pltpu.sync_copy(whs[h], w_hbm[h].at[pl.ds(off, K1)])
            for b in range(NB):
                pltpu.sync_copy(wrows.at[pl.ds(b * 128, 128)],
                                den_sp.at[dstv.at[b]], add=True)
            return 0

        lax.fori_loop(0, EPT // K1, chunk, 0)
        plsc.subcore_barrier()
        pltpu.sync_copy(den_sp.at[pl.ds(sid * RPT, RPT)],
                        dp_hbm.at[cid, pl.ds(sid * RPT, RPT)])

    return pl.kernel(body, out_type=out_type, mesh=_mesh(),
                     scratch_types=scratch,
                     compiler_params=pltpu.CompilerParams(
                         needs_layout_passes=False,
                         use_tc_tiling_on_sc=False))


def _make_b2(H, head_of):
    out_type = [jax.ShapeDtypeStruct((NC, NPAD, LN), jnp.float32)
                for _ in range(4)]
    NB = K2 // 128
    scratch = [pltpu.VMEM((NB, 128), jnp.int32),
               pltpu.VMEM((NB, 128), jnp.int32),
               pltpu.VMEM((K2,), jnp.float32),
               pltpu.VMEM((K2, LN), jnp.float32),
               pltpu.VMEM((ZR, LN), jnp.float32),
               pltpu.VMEM_SHARED((NPAD, LN), jnp.float32),
               pltpu.SemaphoreType.DMA]

    def body(src_hbm, dst_hbm, *rest):
        hs = rest[:4]
        w_hbm = rest[4:4 + H]
        accs = rest[4 + H:8 + H]
        srcv, dstv, wbuf, rows, zbuf, acc_sp, sem = rest[8 + H:]
        cid = lax.axis_index("c")
        sid = lax.axis_index("s")
        wid = cid * NS + sid
        zero16 = jnp.zeros((LN,), jnp.float32)

        def zrow(i, _):
            zbuf[i, :] = zero16
            return 0
        lax.fori_loop(0, ZR, zrow, 0)

        for s in range(4):
            table = hs[s]
            whb = w_hbm[head_of[s]]
            for j in range(RPT // ZR):
                pltpu.sync_copy(zbuf, acc_sp.at[pl.ds(sid * RPT + j * ZR, ZR)])
            plsc.subcore_barrier()

            def chunk(ci, _, table=table, whb=whb):
                off = wid * EPT + ci * K2
                row0 = wid * (EPT // 128) + ci * NB
                pltpu.sync_copy(src_hbm.at[pl.ds(row0, NB)], srcv)
                pltpu.sync_copy(dst_hbm.at[pl.ds(row0, NB)], dstv)
                pltpu.sync_copy(whb.at[pl.ds(off, K2)], wbuf)
                descs = [pltpu.async_copy(
                    table.at[srcv.at[b]], rows.at[pl.ds(b * 128, 128)], sem)
                    for b in range(NB)]
                for d in descs:
                    d.wait()

                def scale(g, _):
                    base = g * 16
                    w16 = wbuf[pl.ds(base, 16)]
                    for j in range(16):
                        rows[base + j, :] = rows[base + j, :] * w16[j]
                    return 0
                lax.fori_loop(0, K2 // 16, scale, 0)
                for b in range(NB):
                    pltpu.sync_copy(rows.at[pl.ds(b * 128, 128)],
                                    acc_sp.at[dstv.at[b]], add=True)
                return 0

            lax.fori_loop(0, EPT // K2, chunk, 0)
            plsc.subcore_barrier()
            pltpu.sync_copy(acc_sp.at[pl.ds(sid * RPT, RPT)],
                            accs[s].at[cid, pl.ds(sid * RPT, RPT)])
            plsc.subcore_barrier()

    return pl.kernel(body, out_type=out_type, mesh=_mesh(),
                     scratch_types=scratch,
                     compiler_params=pltpu.CompilerParams(
                         needs_layout_passes=False,
                         use_tc_tiling_on_sc=False))


def _c1_body(p1_ref, dp1_ref, a0, a1, a2, a3, h0, h1, h2, h3,
             w2_ref, a2s_ref, a2d_ref, b1_ref, h2_ref, p2_ref):
    p1 = p1_ref[...]
    dp = dp1_ref[...]
    accs = (a0, a1, a2, a3)
    hss = (h0, h1, h2, h3)
    rden = []
    for h in range(2):
        den = dp[0][:, h:h + 1] + dp[1][:, h:h + 1] + p1[:, 4 + h:5 + h]
        rden.append(1.0 / (den + 1e-16))
    cols = []
    for s in range(4):
        h = s // 2
        a = accs[s][...]
        tot = a[0] + a[1] + p1[:, 4 + h:5 + h] * hss[s][...]
        cols.append(tot * rden[h] + b1_ref[:, 16 * s:16 * (s + 1)])
    hr = jax.nn.relu(jnp.concatenate(cols, axis=1))
    h2 = jnp.dot(hr, w2_ref[...], preferred_element_type=jnp.float32)
    h2_ref[...] = h2
    as2 = jnp.sum(h2 * a2s_ref[...], axis=1, keepdims=True)
    ad2 = jnp.sum(h2 * a2d_ref[...], axis=1, keepdims=True)
    v = as2 + ad2
    w2s = jnp.exp(jnp.maximum(v, 0.2 * v))
    z = jnp.zeros((RT, 13), jnp.float32)
    p2_ref[...] = jnp.concatenate([as2, ad2, w2s, z], axis=1)


def _c1(p1, dp1, a1accs, h1s, W2, a2s, a2d, b1r):
    full = lambda shp: pl.BlockSpec(shp, lambda i: tuple(0 for _ in shp))
    row = lambda m: pl.BlockSpec((RT, m), lambda i: (i, 0))
    dp_spec = pl.BlockSpec((NC, RT, LN), lambda i: (0, i, 0))
    return pl.pallas_call(
        _c1_body,
        grid=(TGRID,),
        in_specs=([row(LN), dp_spec] + [dp_spec] * 4 + [row(LN)] * 4
                  + [full((64, 64)), full((1, 64)), full((1, 64)),
                     full((1, 64))]),
        out_specs=[row(64), row(LN)],
        out_shape=[jax.ShapeDtypeStruct((NPAD, 64), jnp.float32),
                   jax.ShapeDtypeStruct((NPAD, LN), jnp.float32)],
    )(p1, dp1, *a1accs, *h1s, W2, a2s, a2d, b1r)


def _c2_body(p2_ref, dp2_ref, a0, a1, a2, a3, h2_ref, b2_ref,
             lnw_ref, lnb_ref, o_ref):
    p2 = p2_ref[...]
    dp = dp2_ref[...]
    den = dp[0][:, 0:1] + dp[1][:, 0:1] + p2[:, 2:3]
    rden = 1.0 / (den + 1e-16)
    accs = (a0, a1, a2, a3)
    acc = jnp.concatenate([a[...][0] + a[...][1] for a in accs], axis=1)
    out = (acc + p2[:, 2:3] * h2_ref[...]) * rden + b2_ref[...]
    mu = jnp.mean(out, axis=-1, keepdims=True)
    var = jnp.mean((out - mu) ** 2, axis=-1, keepdims=True)
    o_ref[...] = (out - mu) * lax.rsqrt(var + 1e-5) * lnw_ref[...] + lnb_ref[...]


def _c2(p2, dp2, a2accs, h2, b2r, lnwr, lnbr):
    full = lambda shp: pl.BlockSpec(shp, lambda i: tuple(0 for _ in shp))
    row = lambda m: pl.BlockSpec((RT, m), lambda i: (i, 0))
    dp_spec = pl.BlockSpec((NC, RT, LN), lambda i: (0, i, 0))
    return pl.pallas_call(
        _c2_body,
        grid=(TGRID,),
        in_specs=([row(LN), dp_spec] + [dp_spec] * 4
                  + [row(64), full((1, 64)), full((1, 64)), full((1, 64))]),
        out_specs=row(64),
        out_shape=jax.ShapeDtypeStruct((NPAD, 64), jnp.float32),
    )(p2, dp2, *a2accs, h2, b2r, lnwr, lnbr)


_make_b1 = functools.lru_cache(maxsize=None)(_make_b1)
_make_b2 = functools.lru_cache(maxsize=None)(_make_b2)


def kernel(x, edge_index, type_ids, type_emb, W1, a_src1, a_dst1, b1,
           W2, a_src2, a_dst2, b2, ln_w, ln_b):
    f32 = jnp.float32
    xp = jnp.zeros((NPAD, 8), f32).at[:N, :5].set(x)
    tid = jnp.zeros((NPAD,), jnp.int32).at[:N].set(type_ids)
    oh = (tid[:, None] == jnp.arange(8, dtype=jnp.int32)[None, :]).astype(f32)
    w1a = jnp.zeros((8, 64), f32).at[:5].set(W1[:5])
    w1b = W1[5:]
    a1s = a_src1.reshape(1, 64)
    a1d = a_dst1.reshape(1, 64)
    a2s = a_src2.reshape(1, 64)
    a2d = a_dst2.reshape(1, 64)
    b1r = b1.reshape(1, 64)
    b2r = b2.reshape(1, 64)
    lnwr = ln_w.reshape(1, 64)
    lnbr = ln_b.reshape(1, 64)

    pad_idx = jnp.full((EPAD - E,), N, jnp.int32)
    src2d = jnp.concatenate([edge_index[0], pad_idx]).reshape(EPAD // 128, 128)
    dst2d = jnp.concatenate([edge_index[1], pad_idx]).reshape(EPAD // 128, 128)

    h1, p1 = _a1(xp, oh, type_emb, w1a, w1b, a1s, a1d)
    w10, w11, dp1 = _make_b1(2)(src2d, dst2d, p1)
    h1s = [h1[:, 16 * s:16 * (s + 1)] for s in range(4)]
    a1accs = _make_b2(2, (0, 0, 1, 1))(src2d, dst2d, *h1s, w10, w11)
    h2, p2 = _c1(p1, dp1, a1accs, h1s, W2, a2s, a2d, b1r)
    w20, dp2 = _make_b1(1)(src2d, dst2d, p2)
    h2s = [h2[:, 16 * s:16 * (s + 1)] for s in range(4)]
    a2accs = _make_b2(1, (0, 0, 0, 0))(src2d, dst2d, *h2s, w20)
    out = _c2(p2, dp2, a2accs, h2, b2r, lnwr, lnbr)
    return out[:N]


# speedup vs baseline: 65.6695x; 1.2845x over previous
import functools
import jax
import jax.numpy as jnp
from jax import lax
from jax.experimental import pallas as pl
from jax.experimental.pallas import tpu as pltpu
from jax.experimental.pallas import tpu_sc as plsc

N = 100000
E = 1600000
NC = 2
NS = 16
NW = NC * NS
LN = 16
NPAD = 100352
EPAD = NW * 50176
EPT = EPAD // NW
K1 = 512
K2 = 1024
RPT = NPAD // NS
ZR = 196
RT = 2048
TGRID = NPAD // RT

@functools.lru_cache(maxsize=None)
def _mesh():
    return plsc.VectorSubcoreMesh(core_axis_name="c", subcore_axis_name="s",
                                  num_cores=NC, num_subcores=NS)


def _a1_body(x_ref, oh_ref, temb_ref, w1a_ref, w1b_ref, a1s_ref, a1d_ref,
             h_ref, p_ref):
    te = jnp.dot(oh_ref[...], temb_ref[...], preferred_element_type=jnp.float32)
    h = jnp.dot(x_ref[...], w1a_ref[...], preferred_element_type=jnp.float32)
    h = h + jnp.dot(te, w1b_ref[...], preferred_element_type=jnp.float32)
    h_ref[...] = h
    ts = h * a1s_ref[...]
    td = h * a1d_ref[...]
    as0 = jnp.sum(ts[:, :32], axis=1, keepdims=True)
    as1 = jnp.sum(ts[:, 32:], axis=1, keepdims=True)
    ad0 = jnp.sum(td[:, :32], axis=1, keepdims=True)
    ad1 = jnp.sum(td[:, 32:], axis=1, keepdims=True)
    v0 = as0 + ad0
    v1 = as1 + ad1
    w0 = jnp.exp(jnp.maximum(v0, 0.2 * v0))
    w1 = jnp.exp(jnp.maximum(v1, 0.2 * v1))
    z = jnp.zeros((RT, 10), jnp.float32)
    p_ref[...] = jnp.concatenate([as0, as1, ad0, ad1, w0, w1, z], axis=1)


def _a1(xp, oh, temb, w1a, w1b, a1s, a1d):
    full = lambda shp: pl.BlockSpec(shp, lambda i: tuple(0 for _ in shp))
    row = lambda m: pl.BlockSpec((RT, m), lambda i: (i, 0))
    return pl.pallas_call(
        _a1_body,
        grid=(TGRID,),
        in_specs=[row(8), row(8), full((8, 16)), full((8, 64)),
                  full((16, 64)), full((1, 64)), full((1, 64))],
        out_specs=[row(64), row(LN)],
        out_shape=[jax.ShapeDtypeStruct((NPAD, 64), jnp.float32),
                   jax.ShapeDtypeStruct((NPAD, LN), jnp.float32)],
    )(xp, oh, temb, w1a, w1b, a1s, a1d)


def _make_b1(H):
    KC = 256
    NB = KC // 128
    NCH = EPT // KC
    out_type = ([jax.ShapeDtypeStruct((EPAD,), jnp.float32) for _ in range(H)]
                + [jax.ShapeDtypeStruct((NC, NPAD, LN), jnp.float32)])
    scratch = ([pltpu.VMEM((NB, 128), jnp.int32) for _ in range(4)]
               + [pltpu.VMEM((KC, LN), jnp.float32) for _ in range(4)]
               + [pltpu.VMEM((KC, LN), jnp.float32)]
               + [pltpu.VMEM((KC,), jnp.float32) for _ in range(H)]
               + [pltpu.VMEM((ZR, LN), jnp.float32),
                  pltpu.VMEM_SHARED((NPAD, LN), jnp.float32)]
               + [pltpu.SemaphoreType.DMA for _ in range(4)])

    def body(src_hbm, dst_hbm, p_hbm, *rest):
        w_hbm = rest[:H]
        dp_hbm = rest[H]
        r = rest[H + 1:]
        srcv = r[0:2]
        dstv = r[2:4]
        psrc = r[4:6]
        pdst = r[6:8]
        wrows = r[8]
        whs = r[9:9 + H]
        zbuf, den_sp, si0, si1, sg0, sg1 = r[9 + H:]
        si = (si0, si1)
        sg = (sg0, sg1)
        cid = lax.axis_index("c")
        sid = lax.axis_index("s")
        wid = cid * NS + sid
        zero16 = jnp.zeros((LN,), jnp.float32)
        iota16 = lax.iota(jnp.int32, 16)

        def zrow(i, _):
            zbuf[i, :] = zero16
            return 0
        lax.fori_loop(0, ZR, zrow, 0)

        def wz(i, _):
            wrows[i, :] = zero16
            return 0
        lax.fori_loop(0, KC, wz, 0)

        for j in range(RPT // ZR):
            pltpu.sync_copy(zbuf, den_sp.at[pl.ds(sid * RPT + j * ZR, ZR)])
        plsc.subcore_barrier()

        def idx_start(ci, p):
            row0 = wid * (EPT // 128) + ci * NB
            pltpu.async_copy(src_hbm.at[pl.ds(row0, NB)], srcv[p], si[p])
            pltpu.async_copy(dst_hbm.at[pl.ds(row0, NB)], dstv[p], si[p])

        def idx_wait(ci, p):
            row0 = wid * (EPT // 128) + ci * NB
            pltpu.make_async_copy(src_hbm.at[pl.ds(row0, NB)], srcv[p],
                                  si[p]).wait()
            pltpu.make_async_copy(dst_hbm.at[pl.ds(row0, NB)], dstv[p],
                                  si[p]).wait()

        def g_start(p):
            for b in range(NB):
                pltpu.async_copy(p_hbm.at[srcv[p].at[b]],
                                 psrc[p].at[pl.ds(b * 128, 128)], sg[p])
                pltpu.async_copy(p_hbm.at[dstv[p].at[b]],
                                 pdst[p].at[pl.ds(b * 128, 128)], sg[p])

        def g_wait(p):
            for b in range(NB):
                pltpu.make_async_copy(p_hbm.at[srcv[p].at[b]],
                                      psrc[p].at[pl.ds(b * 128, 128)],
                                      sg[p]).wait()
                pltpu.make_async_copy(p_hbm.at[dstv[p].at[b]],
                                      pdst[p].at[pl.ds(b * 128, 128)],
                                      sg[p]).wait()

        def process(ci, p):
            off = wid * EPT + ci * KC
            for h in range(H):
                colh = jnp.full((16,), h, jnp.int32)
                colad = jnp.full((16,), H + h, jnp.int32)

                def grp(g, _, p=p, h=h, colh=colh, colad=colad):
                    rr = g * 16 + iota16
                    a_s = plsc.load_gather(psrc[p], [rr, colh])
                    a_d = plsc.load_gather(pdst[p], [rr, colad])
                    v = a_s + a_d
                    w = jnp.exp(jnp.maximum(v, 0.2 * v))
                    whs[h][pl.ds(g * 16, 16)] = w
                    plsc.store_scatter(wrows, [rr, colh], w)
                    return 0
                lax.fori_loop(0, KC // 16, grp, 0, unroll=2)
                pltpu.sync_copy(whs[h], w_hbm[h].at[pl.ds(off, KC)])
            for b in range(NB):
                pltpu.sync_copy(wrows.at[pl.ds(b * 128, 128)],
                                den_sp.at[dstv[p].at[b]], add=True)

        idx_start(0, 0)
        idx_wait(0, 0)
        g_start(0)

        def pair(gi, _):
            for p in range(2):
                c = 2 * gi + p
                q = 1 - p

                @pl.when(c + 1 < NCH)
                def _():
                    idx_start(c + 1, q)
                g_wait(p)

                @pl.when(c + 1 < NCH)
                def _():
                    idx_wait(c + 1, q)
                    g_start(q)
                process(c, p)
            return 0
        lax.fori_loop(0, NCH // 2, pair, 0)
        plsc.subcore_barrier()
        pltpu.sync_copy(den_sp.at[pl.ds(sid * RPT, RPT)],
                        dp_hbm.at[cid, pl.ds(sid * RPT, RPT)])

    return pl.kernel(body, out_type=out_type, mesh=_mesh(),
                     scratch_types=scratch,
                     compiler_params=pltpu.CompilerParams(
                         needs_layout_passes=False,
                         use_tc_tiling_on_sc=False))


def _make_b2(H, head_of):
    KC = 512
    NB = KC // 128
    NCH = EPT // KC
    out_type = [jax.ShapeDtypeStruct((NC, NPAD, LN), jnp.float32)
                for _ in range(4)]
    scratch = ([pltpu.VMEM((NB, 128), jnp.int32) for _ in range(4)]
               + [pltpu.VMEM((KC,), jnp.float32) for _ in range(2)]
               + [pltpu.VMEM((KC, LN), jnp.float32) for _ in range(2)]
               + [pltpu.VMEM((ZR, LN), jnp.float32),
                  pltpu.VMEM_SHARED((NPAD, LN), jnp.float32)]
               + [pltpu.SemaphoreType.DMA for _ in range(4)])

    def body(src_hbm, dst_hbm, *rest):
        hs = rest[:4]
        w_hbm = rest[4:4 + H]
        accs = rest[4 + H:8 + H]
        r = rest[8 + H:]
        srcv = r[0:2]
        dstv = r[2:4]
        wb = r[4:6]
        rows = r[6:8]
        zbuf, acc_sp, si0, si1, sg0, sg1 = r[8:]
        si = (si0, si1)
        sg = (sg0, sg1)
        cid = lax.axis_index("c")
        sid = lax.axis_index("s")
        wid = cid * NS + sid
        zero16 = jnp.zeros((LN,), jnp.float32)

        def zrow(i, _):
            zbuf[i, :] = zero16
            return 0
        lax.fori_loop(0, ZR, zrow, 0)

        for s in range(4):
            table = hs[s]
            whb = w_hbm[head_of[s]]

            def idx_start(ci, p, whb=whb):
                off = wid * EPT + ci * KC
                row0 = wid * (EPT // 128) + ci * NB
                pltpu.async_copy(src_hbm.at[pl.ds(row0, NB)], srcv[p], si[p])
                pltpu.async_copy(dst_hbm.at[pl.ds(row0, NB)], dstv[p], si[p])
                pltpu.async_copy(whb.at[pl.ds(off, KC)], wb[p], si[p])

            def idx_wait(ci, p, whb=whb):
                off = wid * EPT + ci * KC
                row0 = wid * (EPT // 128) + ci * NB
                pltpu.make_async_copy(src_hbm.at[pl.ds(row0, NB)], srcv[p],
                                      si[p]).wait()
                pltpu.make_async_copy(dst_hbm.at[pl.ds(row0, NB)], dstv[p],
                                      si[p]).wait()
                pltpu.make_async_copy(whb.at[pl.ds(off, KC)], wb[p],
                                      si[p]).wait()

            def g_start(p, table=table):
                for b in range(NB):
                    pltpu.async_copy(table.at[srcv[p].at[b]],
                                     rows[p].at[pl.ds(b * 128, 128)], sg[p])

            def g_wait(p, table=table):
                for b in range(NB):
                    pltpu.make_async_copy(table.at[srcv[p].at[b]],
                                          rows[p].at[pl.ds(b * 128, 128)],
                                          sg[p]).wait()

            def process(p):
                def scale(g, _, p=p):
                    base = g * 16
                    w16 = wb[p][pl.ds(base, 16)]
                    for j in range(16):
                        rows[p][base + j, :] = rows[p][base + j, :] * w16[j]
                    return 0
                lax.fori_loop(0, KC // 16, scale, 0, unroll=2)
                for b in range(NB):
                    pltpu.sync_copy(rows[p].at[pl.ds(b * 128, 128)],
                                    acc_sp.at[dstv[p].at[b]], add=True)

            for j in range(RPT // ZR):
                pltpu.sync_copy(zbuf, acc_sp.at[pl.ds(sid * RPT + j * ZR, ZR)])
            plsc.subcore_barrier()

            idx_start(0, 0)
            idx_wait(0, 0)
            g_start(0)

            def pair(gi, _):
                for p in range(2):
                    c = 2 * gi + p
                    q = 1 - p

                    @pl.when(c + 1 < NCH)
                    def _():
                        idx_start(c + 1, q)
                    g_wait(p)

                    @pl.when(c + 1 < NCH)
                    def _():
                        idx_wait(c + 1, q)
                        g_start(q)
                    process(p)
                return 0
            lax.fori_loop(0, NCH // 2, pair, 0)
            plsc.subcore_barrier()
            pltpu.sync_copy(acc_sp.at[pl.ds(sid * RPT, RPT)],
                            accs[s].at[cid, pl.ds(sid * RPT, RPT)])
            plsc.subcore_barrier()

    return pl.kernel(body, out_type=out_type, mesh=_mesh(),
                     scratch_types=scratch,
                     compiler_params=pltpu.CompilerParams(
                         needs_layout_passes=False,
                         use_tc_tiling_on_sc=False))


def _c1_body(p1_ref, dp1_ref, a0, a1, a2, a3, h0, h1, h2, h3,
             w2_ref, a2s_ref, a2d_ref, b1_ref, h2_ref, p2_ref):
    p1 = p1_ref[...]
    dp = dp1_ref[...]
    accs = (a0, a1, a2, a3)
    hss = (h0, h1, h2, h3)
    rden = []
    for h in range(2):
        den = dp[0][:, h:h + 1] + dp[1][:, h:h + 1] + p1[:, 4 + h:5 + h]
        rden.append(1.0 / (den + 1e-16))
    cols = []
    for s in range(4):
        h = s // 2
        a = accs[s][...]
        tot = a[0] + a[1] + p1[:, 4 + h:5 + h] * hss[s][...]
        cols.append(tot * rden[h] + b1_ref[:, 16 * s:16 * (s + 1)])
    hr = jax.nn.relu(jnp.concatenate(cols, axis=1))
    h2 = jnp.dot(hr, w2_ref[...], preferred_element_type=jnp.float32)
    h2_ref[...] = h2
    as2 = jnp.sum(h2 * a2s_ref[...], axis=1, keepdims=True)
    ad2 = jnp.sum(h2 * a2d_ref[...], axis=1, keepdims=True)
    v = as2 + ad2
    w2s = jnp.exp(jnp.maximum(v, 0.2 * v))
    z = jnp.zeros((RT, 13), jnp.float32)
    p2_ref[...] = jnp.concatenate([as2, ad2, w2s, z], axis=1)


def _c1(p1, dp1, a1accs, h1s, W2, a2s, a2d, b1r):
    full = lambda shp: pl.BlockSpec(shp, lambda i: tuple(0 for _ in shp))
    row = lambda m: pl.BlockSpec((RT, m), lambda i: (i, 0))
    dp_spec = pl.BlockSpec((NC, RT, LN), lambda i: (0, i, 0))
    return pl.pallas_call(
        _c1_body,
        grid=(TGRID,),
        in_specs=([row(LN), dp_spec] + [dp_spec] * 4 + [row(LN)] * 4
                  + [full((64, 64)), full((1, 64)), full((1, 64)),
                     full((1, 64))]),
        out_specs=[row(64), row(LN)],
        out_shape=[jax.ShapeDtypeStruct((NPAD, 64), jnp.float32),
                   jax.ShapeDtypeStruct((NPAD, LN), jnp.float32)],
    )(p1, dp1, *a1accs, *h1s, W2, a2s, a2d, b1r)


def _c2_body(p2_ref, dp2_ref, a0, a1, a2, a3, h2_ref, b2_ref,
             lnw_ref, lnb_ref, o_ref):
    p2 = p2_ref[...]
    dp = dp2_ref[...]
    den = dp[0][:, 0:1] + dp[1][:, 0:1] + p2[:, 2:3]
    rden = 1.0 / (den + 1e-16)
    accs = (a0, a1, a2, a3)
    acc = jnp.concatenate([a[...][0] + a[...][1] for a in accs], axis=1)
    out = (acc + p2[:, 2:3] * h2_ref[...]) * rden + b2_ref[...]
    mu = jnp.mean(out, axis=-1, keepdims=True)
    var = jnp.mean((out - mu) ** 2, axis=-1, keepdims=True)
    o_ref[...] = (out - mu) * lax.rsqrt(var + 1e-5) * lnw_ref[...] + lnb_ref[...]


def _c2(p2, dp2, a2accs, h2, b2r, lnwr, lnbr):
    full = lambda shp: pl.BlockSpec(shp, lambda i: tuple(0 for _ in shp))
    row = lambda m: pl.BlockSpec((RT, m), lambda i: (i, 0))
    dp_spec = pl.BlockSpec((NC, RT, LN), lambda i: (0, i, 0))
    return pl.pallas_call(
        _c2_body,
        grid=(TGRID,),
        in_specs=([row(LN), dp_spec] + [dp_spec] * 4
                  + [row(64), full((1, 64)), full((1, 64)), full((1, 64))]),
        out_specs=row(64),
        out_shape=jax.ShapeDtypeStruct((NPAD, 64), jnp.float32),
    )(p2, dp2, *a2accs, h2, b2r, lnwr, lnbr)


_make_b1 = functools.lru_cache(maxsize=None)(_make_b1)
_make_b2 = functools.lru_cache(maxsize=None)(_make_b2)


def kernel(x, edge_index, type_ids, type_emb, W1, a_src1, a_dst1, b1,
           W2, a_src2, a_dst2, b2, ln_w, ln_b):
    f32 = jnp.float32
    xp = jnp.zeros((NPAD, 8), f32).at[:N, :5].set(x)
    tid = jnp.zeros((NPAD,), jnp.int32).at[:N].set(type_ids)
    oh = (tid[:, None] == jnp.arange(8, dtype=jnp.int32)[None, :]).astype(f32)
    w1a = jnp.zeros((8, 64), f32).at[:5].set(W1[:5])
    w1b = W1[5:]
    a1s = a_src1.reshape(1, 64)
    a1d = a_dst1.reshape(1, 64)
    a2s = a_src2.reshape(1, 64)
    a2d = a_dst2.reshape(1, 64)
    b1r = b1.reshape(1, 64)
    b2r = b2.reshape(1, 64)
    lnwr = ln_w.reshape(1, 64)
    lnbr = ln_b.reshape(1, 64)

    pad_idx = jnp.full((EPAD - E,), N, jnp.int32)
    src2d = jnp.concatenate([edge_index[0], pad_idx]).reshape(EPAD // 128, 128)
    dst2d = jnp.concatenate([edge_index[1], pad_idx]).reshape(EPAD // 128, 128)

    h1, p1 = _a1(xp, oh, type_emb, w1a, w1b, a1s, a1d)
    w10, w11, dp1 = _make_b1(2)(src2d, dst2d, p1)
    h1s = [h1[:, 16 * s:16 * (s + 1)] for s in range(4)]
    a1accs = _make_b2(2, (0, 0, 1, 1))(src2d, dst2d, *h1s, w10, w11)
    h2, p2 = _c1(p1, dp1, a1accs, h1s, W2, a2s, a2d, b1r)
    w20, dp2 = _make_b1(1)(src2d, dst2d, p2)
    h2s = [h2[:, 16 * s:16 * (s + 1)] for s in range(4)]
    a2accs = _make_b2(1, (0, 0, 0, 0))(src2d, dst2d, *h2s, w20)
    out = _c2(p2, dp2, a2accs, h2, b2r, lnwr, lnbr)
    return out[:N]
